# Initial kernel scaffold; baseline (speedup 1.0000x reference)
import jax
import jax.numpy as jnp
from jax.experimental import pallas as pl


def kernel(abstract, emb_table, W, b):
    raise NotImplementedError("write your pallas kernel here")


# speedup vs baseline: 4.1228x; 4.1228x over previous
import functools

import jax
import jax.numpy as jnp
from jax import lax
from jax.experimental import pallas as pl
from jax.experimental.pallas import tpu as pltpu
from jax.experimental.pallas import tpu_sc as plsc

B, L, H, OUT, V = 4096, 50, 64, 6, 100000
NC, NS = 2, 16
NW = NC * NS
BPW = B // NW
NB = 4
NG = BPW // NB
HC = H // 16


def _sc_body(abs_hbm, tab_hbm, w_hbm, bias_hbm, out_hbm,
             idx_v, w_v, rows_v, out_v, bias_v, tbuf_v, sem):
    wid = lax.axis_index("s") * NC + lax.axis_index("c")
    base = wid * BPW
    pltpu.sync_copy(abs_hbm.at[pl.ds(base, BPW), :], idx_v)
    pltpu.sync_copy(w_hbm, w_v)
    pltpu.sync_copy(bias_hbm, bias_v)
    lane = lax.iota(jnp.int32, 16)
    col_idx = [lane * 16 + o for o in range(OUT)]

    def group(g, carry):
        descs = [
            pltpu.async_copy(tab_hbm.at[idx_v.at[g * NB + b]], rows_v.at[b], sem)
            for b in range(NB)
        ]
        for d in descs:
            d.wait()

        def l_body(l, accs):
            rows = [rows_v[b, l, pl.ds(c * 16, 16)]
                    for b in range(NB) for c in range(HC)]
            new = list(accs)
            for o in range(OUT):
                w = [w_v[l, o, pl.ds(c * 16, 16)] for c in range(HC)]
                for b in range(NB):
                    a = new[b * OUT + o]
                    for c in range(HC):
                        a = a + w[c] * rows[b * HC + c]
                    new[b * OUT + o] = a
            return tuple(new)

        zeros = tuple(jnp.zeros((16,), jnp.float32) for _ in range(NB * OUT))
        accs = lax.fori_loop(0, L, l_body, zeros)

        bias = bias_v[:]
        for b in range(NB):
            for o in range(OUT):
                plsc.store_scatter(tbuf_v, [col_idx[o]], accs[b * OUT + o])
            row = bias
            for h in range(16):
                row = row + tbuf_v[pl.ds(h * 16, 16)]
            out_v[g * NB + b, :] = row
        return carry

    lax.fori_loop(0, NG, group, 0)
    pltpu.sync_copy(out_v, out_hbm.at[pl.ds(base, BPW), :])


@jax.jit
def kernel(abstract, emb_table, W, b):
    wt = W.reshape(OUT, L, H).transpose(1, 0, 2)
    bias16 = jnp.zeros((16,), jnp.float32).at[:OUT].set(b)
    mesh = plsc.VectorSubcoreMesh(
        core_axis_name="c", subcore_axis_name="s",
        num_cores=NC, num_subcores=NS)
    f = pl.kernel(
        _sc_body,
        out_type=jax.ShapeDtypeStruct((B, 16), jnp.float32),
        mesh=mesh,
        compiler_params=pltpu.CompilerParams(
            needs_layout_passes=False, use_tc_tiling_on_sc=False),
        scratch_types=[
            pltpu.VMEM((BPW, L), jnp.int32),
            pltpu.VMEM((L, OUT, H), jnp.float32),
            pltpu.VMEM((NB, L, H), jnp.float32),
            pltpu.VMEM((BPW, 16), jnp.float32),
            pltpu.VMEM((16,), jnp.float32),
            pltpu.VMEM((256,), jnp.float32),
            pltpu.SemaphoreType.DMA,
        ],
    )
    out = f(abstract, emb_table, wt, bias16)
    return out[:, :OUT]


# speedup vs baseline: 4.9649x; 1.2043x over previous
import functools

import jax
import jax.numpy as jnp
from jax import lax
from jax.experimental import pallas as pl
from jax.experimental.pallas import tpu as pltpu
from jax.experimental.pallas import tpu_sc as plsc

B, L, H, OUT, V = 4096, 50, 64, 6, 100000
NC, NS = 2, 16
NW = NC * NS
BPW = B // NW
NB = 4
NG = BPW // NB
HC = H // 16


def _sc_body(abs_hbm, tab_hbm, w_hbm, bias_hbm, out_hbm,
             idx_v, w_v, rows_v, out_v, bias_v, tbuf_v, sems):
    wid = lax.axis_index("s") * NC + lax.axis_index("c")
    base = wid * BPW
    pltpu.sync_copy(abs_hbm.at[pl.ds(base, BPW), :], idx_v)
    pltpu.sync_copy(w_hbm, w_v)
    pltpu.sync_copy(bias_hbm, bias_v)
    lane = lax.iota(jnp.int32, 16)
    col_idx = [lane * 16 + o for o in range(OUT)]

    def fire(g, phase):
        for b in range(NB):
            pltpu.async_copy(tab_hbm.at[idx_v.at[g * NB + b]],
                             rows_v.at[phase * NB + b], sems.at[phase])

    def drain(g, phase):
        for b in range(NB):
            pltpu.make_async_copy(tab_hbm.at[idx_v.at[g * NB + b]],
                                  rows_v.at[phase * NB + b],
                                  sems.at[phase]).wait()

    fire(0, 0)
    fire(1, 1)

    def pair(i, carry):
        for phase in range(2):
            g = i * 2 + phase
            drain(g, phase)

            def l_body(l, accs):
                rows = [rows_v[phase * NB + b, l, pl.ds(c * 16, 16)]
                        for b in range(NB) for c in range(HC)]
                new = list(accs)
                for o in range(OUT):
                    w = [w_v[l, o, pl.ds(c * 16, 16)] for c in range(HC)]
                    for b in range(NB):
                        a = new[b * OUT + o]
                        for c in range(HC):
                            a = a + w[c] * rows[b * HC + c]
                        new[b * OUT + o] = a
                return tuple(new)

            zeros = tuple(jnp.zeros((16,), jnp.float32)
                          for _ in range(NB * OUT))
            accs = lax.fori_loop(0, L, l_body, zeros)

            @pl.when(g + 2 < NG)
            def _():
                fire(g + 2, phase)

            bias = bias_v[:]
            for b in range(NB):
                for o in range(OUT):
                    plsc.store_scatter(tbuf_v, [col_idx[o]], accs[b * OUT + o])
                row = bias
                for h in range(16):
                    row = row + tbuf_v[pl.ds(h * 16, 16)]
                out_v[g * NB + b, :] = row
        return carry

    lax.fori_loop(0, NG // 2, pair, 0)
    pltpu.sync_copy(out_v, out_hbm.at[pl.ds(base, BPW), :])


@jax.jit
def kernel(abstract, emb_table, W, b):
    wt = W.reshape(OUT, L, H).transpose(1, 0, 2)
    bias16 = jnp.zeros((16,), jnp.float32).at[:OUT].set(b)
    mesh = plsc.VectorSubcoreMesh(
        core_axis_name="c", subcore_axis_name="s",
        num_cores=NC, num_subcores=NS)
    f = pl.kernel(
        _sc_body,
        out_type=jax.ShapeDtypeStruct((B, 16), jnp.float32),
        mesh=mesh,
        compiler_params=pltpu.CompilerParams(
            needs_layout_passes=False, use_tc_tiling_on_sc=False),
        scratch_types=[
            pltpu.VMEM((BPW, L), jnp.int32),
            pltpu.VMEM((L, OUT, H), jnp.float32),
            pltpu.VMEM((2 * NB, L, H), jnp.float32),
            pltpu.VMEM((BPW, 16), jnp.float32),
            pltpu.VMEM((16,), jnp.float32),
            pltpu.VMEM((256,), jnp.float32),
            pltpu.SemaphoreType.DMA((2,)),
        ],
    )
    out = f(abstract, emb_table, wt, bias16)
    return out[:, :OUT]


# speedup vs baseline: 5.4853x; 1.1048x over previous
import functools

import jax
import jax.numpy as jnp
from jax import lax
from jax.experimental import pallas as pl
from jax.experimental.pallas import tpu as pltpu
from jax.experimental.pallas import tpu_sc as plsc

B, L, H, OUT, V = 4096, 50, 64, 6, 100000
NC, NS = 2, 16
NW = NC * NS
BPW = B // NW
NB = 4
NG = BPW // NB
HC = H // 16
HC2 = H // 32
KF = 5


def _sc_body(abs_hbm, tab_hbm, w_hbm, bias_hbm, out_hbm,
             idx_v, w_v, rows_v, out_v, bias_v, tbuf_v, sems):
    wid = lax.axis_index("s") * NC + lax.axis_index("c")
    base = wid * BPW
    pltpu.sync_copy(abs_hbm.at[pl.ds(base, BPW), :], idx_v)
    pltpu.sync_copy(w_hbm, w_v)
    pltpu.sync_copy(bias_hbm, bias_v)
    lane = lax.iota(jnp.int32, 16)
    col_idx = [lane * 16 + o for o in range(OUT)]

    def fire(g, phase):
        for b in range(NB):
            pltpu.async_copy(tab_hbm.at[idx_v.at[g * NB + b]],
                             rows_v.at[phase * NB + b], sems.at[phase])

    def drain(g, phase):
        for b in range(NB):
            pltpu.make_async_copy(tab_hbm.at[idx_v.at[g * NB + b]],
                                  rows_v.at[phase * NB + b],
                                  sems.at[phase]).wait()

    fire(0, 0)
    fire(1, 1)

    def pair(i, carry):
        for phase in range(2):
            g = i * 2 + phase
            drain(g, phase)

            def step_body(s, faccs):
                accs = [jnp.zeros((32,), jnp.bfloat16)
                        for _ in range(NB * OUT)]
                for dl in range(KF):
                    l = s * KF + dl
                    rows = [rows_v[phase * NB + b, l, pl.ds(c * 32, 32)]
                            for b in range(NB) for c in range(HC2)]
                    for o in range(OUT):
                        w = [w_v[l, o, pl.ds(c * 32, 32)]
                             for c in range(HC2)]
                        for b in range(NB):
                            a = accs[b * OUT + o]
                            for c in range(HC2):
                                a = a + w[c] * rows[b * HC2 + c]
                            accs[b * OUT + o] = a
                new = list(faccs)
                for i in range(NB * OUT):
                    p0, p1 = plsc.unpack(
                        accs[i], format=plsc.PackFormat.INTERLEAVED,
                        preferred_element_type=jnp.float32)
                    new[i] = new[i] + (p0 + p1)
                return tuple(new)

            fzeros = tuple(jnp.zeros((16,), jnp.float32)
                           for _ in range(NB * OUT))
            faccs = lax.fori_loop(0, L // KF, step_body, fzeros)

            @pl.when(g + 2 < NG)
            def _():
                fire(g + 2, phase)

            bias = bias_v[:]
            for b in range(NB):
                for o in range(OUT):
                    plsc.store_scatter(tbuf_v, [col_idx[o]],
                                       faccs[b * OUT + o])
                row = bias
                for h in range(16):
                    row = row + tbuf_v[pl.ds(h * 16, 16)]
                out_v[g * NB + b, :] = row
        return carry

    lax.fori_loop(0, NG // 2, pair, 0)
    pltpu.sync_copy(out_v, out_hbm.at[pl.ds(base, BPW), :])


@jax.jit
def kernel(abstract, emb_table, W, b):
    tab16 = emb_table.astype(jnp.bfloat16)
    wt = W.reshape(OUT, L, H).transpose(1, 0, 2).astype(jnp.bfloat16)
    bias16 = jnp.zeros((16,), jnp.float32).at[:OUT].set(b)
    mesh = plsc.VectorSubcoreMesh(
        core_axis_name="c", subcore_axis_name="s",
        num_cores=NC, num_subcores=NS)
    f = pl.kernel(
        _sc_body,
        out_type=jax.ShapeDtypeStruct((B, 16), jnp.float32),
        mesh=mesh,
        compiler_params=pltpu.CompilerParams(
            needs_layout_passes=False, use_tc_tiling_on_sc=False),
        scratch_types=[
            pltpu.VMEM((BPW, L), jnp.int32),
            pltpu.VMEM((L, OUT, H), jnp.bfloat16),
            pltpu.VMEM((2 * NB, L, H), jnp.bfloat16),
            pltpu.VMEM((BPW, 16), jnp.float32),
            pltpu.VMEM((16,), jnp.float32),
            pltpu.VMEM((256,), jnp.float32),
            pltpu.SemaphoreType.DMA((2,)),
        ],
    )
    out = f(abstract, tab16, wt, bias16)
    return out[:, :OUT]
